# baseline (speedup 1.0000x reference)
import functools

import jax
import jax.numpy as jnp
from jax import lax
from jax.experimental import pallas as pl
from jax.experimental.pallas import tpu as pltpu
from jax.experimental.pallas import tpu_sc as plsc

_PART_FRACS = (0.4, 0.3, 0.3)
_F32_ONE_BITS = 0x3F800000


def _part_bounds(h):
    bounds = []
    start = 0
    for f in _PART_FRACS:
        end = min(start + int(h * f), h)
        bounds.append((start, end))
        start = end
    return bounds


def _pool_body(x_ref, out_ref, *, bounds, w, hw):
    xb = x_ref[0]
    e = lax.broadcasted_iota(jnp.int32, (4, hw), 1)
    row = e // w
    s = lax.broadcasted_iota(jnp.int32, (4, hw), 0)
    m = (s == 3) & (row >= bounds[2][1])
    for i, (lo, hi) in enumerate(bounds):
        m = m | ((s == i) & (row >= lo) & (row < hi))
    mask = m.astype(jnp.float32)
    out_ref[0] = lax.dot_general(
        mask, xb, (((1,), (0,)), ((), ())),
        precision=lax.Precision.HIGHEST, preferred_element_type=jnp.float32)


def _att_body(sums_ref, W1_ref, b1_ref, W2_ref, b2_ref, Wp_ref, bp_ref,
              G1_ref, g1_ref, G2_ref, g2_ref, fa_ref, pw_ref, k_ref,
              *, bounds, h, w):
    c = fa_ref.shape[1]
    sums = sums_ref[...]
    gp = ((sums[:, 0, :] + sums[:, 1, :] + sums[:, 2, :] + sums[:, 3, :])
          * (1.0 / (h * w)))

    def dot_t(a, b):
        return lax.dot_general(a, b, (((1,), (1,)), ((), ())),
                               precision=lax.Precision.DEFAULT,
                               preferred_element_type=jnp.float32)

    atts = []
    for i, (lo, hi) in enumerate(bounds):
        pooled = sums[:, i, :] * (1.0 / ((hi - lo) * w))
        hdn = jax.nn.relu(dot_t(pooled, W1_ref[i]) + b1_ref[i:i + 1, :])
        atts.append(jax.nn.sigmoid(dot_t(hdn, W2_ref[i]) + b2_ref[i:i + 1, :]))

    logits = dot_t(gp, Wp_ref[...]) + bp_ref[...]
    mx = jnp.max(logits, axis=1, keepdims=True)
    ex = jnp.exp(logits - mx)
    pw = ex / jnp.sum(ex, axis=1, keepdims=True)

    fused = (pw[:, 0:1] * atts[0] + pw[:, 1:2] * atts[1]
             + pw[:, 2:3] * atts[2])

    hg = jax.nn.relu(dot_t(gp, G1_ref[...]) + g1_ref[...])
    sp_logit = jnp.sum(hg * G2_ref[...], axis=1, keepdims=True)
    sp = jax.nn.sigmoid(sp_logit + g2_ref[...])
    k = jnp.clip((sp * c).astype(jnp.int32), 1, c)

    fa_ref[...] = lax.bitcast_convert_type(fused, jnp.int32)
    pw_ref[...] = pw
    k_ref[...] = k


def _topk_sc_body(fused_hbm, k_hbm, fa_hbm, row_v, out_v, k_v, *, c, rows):
    nchunk = c // 16
    wid = lax.axis_index("s") * 2 + lax.axis_index("c")
    pltpu.sync_copy(k_hbm, k_v)

    def count_gt(t):
        tv = jnp.full((16,), t, jnp.int32)

        def cbody(i, acc):
            bb = row_v[0, pl.ds(i * 16, 16)]
            return acc + (bb > tv).astype(jnp.int32)

        perlane = lax.fori_loop(0, nchunk, cbody,
                                jnp.zeros((16,), jnp.int32))
        return jnp.sum(perlane)

    for j in range(rows):
        r = wid * rows + j
        pltpu.sync_copy(fused_hbm.at[pl.ds(r, 1)], row_v)
        base = (r // 16) * 16
        kvec = k_v[pl.ds(base, 16)]
        lane = lax.broadcasted_iota(jnp.int32, (16,), 0)
        kk = jnp.sum(jnp.where(lane == jnp.full((16,), r - base, jnp.int32),
                               kvec, 0))

        def vbody(_, carry):
            lo, hi = carry
            mid = lax.shift_right_logical(lo + hi, 1)
            pred = count_gt(mid) < kk
            return (jnp.where(pred, lo, mid + 1),
                    jnp.where(pred, mid, hi))

        lo, hi = lax.fori_loop(0, 31, vbody,
                               (jnp.int32(0), jnp.int32(_F32_ONE_BITS)))
        t = lo
        m = kk - count_gt(t)
        tv = jnp.full((16,), t, jnp.int32)
        mv = jnp.full((16,), m, jnp.int32)

        def wbody(i, tacc):
            bb = row_v[0, pl.ds(i * 16, 16)]
            eq = bb == tv
            eqi = eq.astype(jnp.int32)
            before = plsc.cumsum(eqi) - eqi
            sel = (bb > tv) | (eq & ((before + jnp.full((16,), tacc,
                                                        jnp.int32)) < mv))
            out_v[0, pl.ds(i * 16, 16)] = jnp.where(sel, bb, 0)
            return tacc + jnp.sum(eqi)

        lax.fori_loop(0, nchunk, wbody, jnp.int32(0))
        pltpu.sync_copy(out_v, fa_hbm.at[pl.ds(r, 1)])


def _apply_body(x_ref, fa_ref, out1_ref, out2_ref):
    xb = x_ref[0]
    f = fa_ref[0]
    o1 = xb * f
    out1_ref[0] = o1
    out2_ref[0] = xb - o1


def kernel(x, W1, b1, W2, b2, Wp, bp, G1, g1, G2, g2):
    b, c, h, w = x.shape
    hw = h * w
    bounds = _part_bounds(h)
    xcl = x.transpose(0, 2, 3, 1).reshape(b, hw, c)

    sums = pl.pallas_call(
        functools.partial(_pool_body, bounds=bounds, w=w, hw=hw),
        grid=(b,),
        in_specs=[pl.BlockSpec((1, hw, c), lambda i: (i, 0, 0))],
        out_specs=pl.BlockSpec((1, 4, c), lambda i: (i, 0, 0)),
        out_shape=jax.ShapeDtypeStruct((b, 4, c), jnp.float32),
    )(xcl)

    fused_bits, pw, k = pl.pallas_call(
        functools.partial(_att_body, bounds=bounds, h=h, w=w),
        out_shape=(
            jax.ShapeDtypeStruct((b, c), jnp.int32),
            jax.ShapeDtypeStruct((b, 3), jnp.float32),
            jax.ShapeDtypeStruct((b, 1), jnp.int32),
        ),
    )(sums, W1, b1, W2, b2, Wp, bp.reshape(1, 3), G1,
      g1.reshape(1, -1), G2, jnp.broadcast_to(g2.reshape(1, 1), (b, 1)))

    mesh = plsc.VectorSubcoreMesh(core_axis_name="c", subcore_axis_name="s")
    rows_per_worker = b // 32
    fa_bits = pl.kernel(
        functools.partial(_topk_sc_body, c=c, rows=rows_per_worker),
        mesh=mesh,
        out_type=jax.ShapeDtypeStruct((b, c), jnp.int32),
        scratch_types=[
            pltpu.VMEM((1, c), jnp.int32),
            pltpu.VMEM((1, c), jnp.int32),
            pltpu.VMEM((b,), jnp.int32),
        ],
        compiler_params=pltpu.CompilerParams(needs_layout_passes=False),
    )(fused_bits, k.reshape(b))
    fa = lax.bitcast_convert_type(fa_bits, jnp.float32)

    out1, out2 = pl.pallas_call(
        _apply_body,
        grid=(b,),
        in_specs=[
            pl.BlockSpec((1, hw, c), lambda i: (i, 0, 0)),
            pl.BlockSpec((1, 1, c), lambda i: (i, 0, 0)),
        ],
        out_specs=[
            pl.BlockSpec((1, hw, c), lambda i: (i, 0, 0)),
            pl.BlockSpec((1, hw, c), lambda i: (i, 0, 0)),
        ],
        out_shape=(
            jax.ShapeDtypeStruct((b, hw, c), jnp.float32),
            jax.ShapeDtypeStruct((b, hw, c), jnp.float32),
        ),
    )(xcl, fa.reshape(b, 1, c))

    def back(o):
        return o.reshape(b, h, w, c).transpose(0, 3, 1, 2)

    return back(out1), back(out2), pw.reshape(b, 3, 1, 1)


# speedup vs baseline: 1.1136x; 1.1136x over previous
import functools

import jax
import jax.numpy as jnp
from jax import lax
from jax.experimental import pallas as pl
from jax.experimental.pallas import tpu as pltpu
from jax.experimental.pallas import tpu_sc as plsc

_PART_FRACS = (0.4, 0.3, 0.3)
_F32_ONE_BITS = 0x3F800000


def _part_bounds(h):
    bounds = []
    start = 0
    for f in _PART_FRACS:
        end = min(start + int(h * f), h)
        bounds.append((start, end))
        start = end
    return bounds


def _pool_body(x_ref, out_ref, *, bounds, w, hw):
    xb = x_ref[0]
    e = lax.broadcasted_iota(jnp.int32, (4, hw), 1)
    row = e // w
    s = lax.broadcasted_iota(jnp.int32, (4, hw), 0)
    m = (s == 3) & (row >= bounds[2][1])
    for i, (lo, hi) in enumerate(bounds):
        m = m | ((s == i) & (row >= lo) & (row < hi))
    mask = m.astype(jnp.float32)
    out_ref[0] = lax.dot_general(
        mask, xb, (((1,), (0,)), ((), ())),
        precision=lax.Precision.HIGHEST, preferred_element_type=jnp.float32)


def _att_body(sums_ref, W1_ref, b1_ref, W2_ref, b2_ref, Wp_ref, bp_ref,
              G1_ref, g1_ref, G2_ref, g2_ref, fa_ref, pw_ref, k_ref,
              *, bounds, h, w):
    c = fa_ref.shape[1]
    sums = sums_ref[...]
    gp = ((sums[:, 0, :] + sums[:, 1, :] + sums[:, 2, :] + sums[:, 3, :])
          * (1.0 / (h * w)))

    def dot_t(a, b):
        return lax.dot_general(a, b, (((1,), (1,)), ((), ())),
                               precision=lax.Precision.DEFAULT,
                               preferred_element_type=jnp.float32)

    atts = []
    for i, (lo, hi) in enumerate(bounds):
        pooled = sums[:, i, :] * (1.0 / ((hi - lo) * w))
        hdn = jax.nn.relu(dot_t(pooled, W1_ref[i]) + b1_ref[i:i + 1, :])
        atts.append(jax.nn.sigmoid(dot_t(hdn, W2_ref[i]) + b2_ref[i:i + 1, :]))

    logits = dot_t(gp, Wp_ref[...]) + bp_ref[...]
    mx = jnp.max(logits, axis=1, keepdims=True)
    ex = jnp.exp(logits - mx)
    pw = ex / jnp.sum(ex, axis=1, keepdims=True)

    fused = (pw[:, 0:1] * atts[0] + pw[:, 1:2] * atts[1]
             + pw[:, 2:3] * atts[2])

    hg = jax.nn.relu(dot_t(gp, G1_ref[...]) + g1_ref[...])
    sp_logit = jnp.sum(hg * G2_ref[...], axis=1, keepdims=True)
    sp = jax.nn.sigmoid(sp_logit + g2_ref[...])
    k = jnp.clip((sp * c).astype(jnp.int32), 1, c)

    fa_ref[...] = lax.bitcast_convert_type(fused, jnp.int32)
    pw_ref[...] = pw
    k_ref[...] = k


def _topk_sc_body(fused_hbm, k_hbm, fa_hbm, row_v, out_v, k_v, *, c, rows):
    nchunk = c // 16
    wid = lax.axis_index("s") * 2 + lax.axis_index("c")
    pltpu.sync_copy(k_hbm, k_v)

    UN = 8

    def count_gt(t):
        tv = jnp.full((16,), t, jnp.int32)

        def cbody(i, acc):
            for u in range(UN):
                bb = row_v[0, pl.ds((i * UN + u) * 16, 16)]
                acc = acc + (bb > tv).astype(jnp.int32)
            return acc

        perlane = lax.fori_loop(0, nchunk // UN, cbody,
                                jnp.zeros((16,), jnp.int32))
        return jnp.sum(perlane)

    def row_min_max():
        def mb(i, carry):
            mn, mx = carry
            for u in range(UN):
                bb = row_v[0, pl.ds((i * UN + u) * 16, 16)]
                mn = jnp.minimum(mn, bb)
                mx = jnp.maximum(mx, bb)
            return mn, mx

        mn, mx = lax.fori_loop(
            0, nchunk // UN, mb,
            (jnp.full((16,), _F32_ONE_BITS, jnp.int32),
             jnp.zeros((16,), jnp.int32)))
        return jnp.min(mn), jnp.max(mx)

    for j in range(rows):
        r = wid * rows + j
        pltpu.sync_copy(fused_hbm.at[pl.ds(r, 1)], row_v)
        base = (r // 16) * 16
        kvec = k_v[pl.ds(base, 16)]
        lane = lax.broadcasted_iota(jnp.int32, (16,), 0)
        kk = jnp.sum(jnp.where(lane == jnp.full((16,), r - base, jnp.int32),
                               kvec, 0))

        def vcond(carry):
            lo, hi = carry
            return lo < hi

        def vbody(carry):
            lo, hi = carry
            mid = lax.shift_right_logical(lo + hi, 1)
            pred = count_gt(mid) < kk
            return (jnp.where(pred, lo, mid + 1),
                    jnp.where(pred, mid, hi))

        mn, mx = row_min_max()
        lo, hi = lax.while_loop(vcond, vbody, (mn, mx))
        t = lo
        m = kk - count_gt(t)
        tv = jnp.full((16,), t, jnp.int32)
        mv = jnp.full((16,), m, jnp.int32)

        def wbody(i, tacc):
            bb = row_v[0, pl.ds(i * 16, 16)]
            eq = bb == tv
            eqi = eq.astype(jnp.int32)
            before = plsc.cumsum(eqi) - eqi
            sel = (bb > tv) | (eq & ((before + jnp.full((16,), tacc,
                                                        jnp.int32)) < mv))
            out_v[0, pl.ds(i * 16, 16)] = jnp.where(sel, bb, 0)
            return tacc + jnp.sum(eqi)

        lax.fori_loop(0, nchunk, wbody, jnp.int32(0))
        pltpu.sync_copy(out_v, fa_hbm.at[pl.ds(r, 1)])


def _apply_body(x_ref, fa_ref, out1_ref, out2_ref):
    xb = x_ref[0]
    f = fa_ref[0]
    o1 = xb * f
    out1_ref[0] = o1
    out2_ref[0] = xb - o1


def kernel(x, W1, b1, W2, b2, Wp, bp, G1, g1, G2, g2):
    b, c, h, w = x.shape
    hw = h * w
    bounds = _part_bounds(h)
    xcl = x.transpose(0, 2, 3, 1).reshape(b, hw, c)

    sums = pl.pallas_call(
        functools.partial(_pool_body, bounds=bounds, w=w, hw=hw),
        grid=(b,),
        in_specs=[pl.BlockSpec((1, hw, c), lambda i: (i, 0, 0))],
        out_specs=pl.BlockSpec((1, 4, c), lambda i: (i, 0, 0)),
        out_shape=jax.ShapeDtypeStruct((b, 4, c), jnp.float32),
    )(xcl)

    fused_bits, pw, k = pl.pallas_call(
        functools.partial(_att_body, bounds=bounds, h=h, w=w),
        out_shape=(
            jax.ShapeDtypeStruct((b, c), jnp.int32),
            jax.ShapeDtypeStruct((b, 3), jnp.float32),
            jax.ShapeDtypeStruct((b, 1), jnp.int32),
        ),
    )(sums, W1, b1, W2, b2, Wp, bp.reshape(1, 3), G1,
      g1.reshape(1, -1), G2, jnp.broadcast_to(g2.reshape(1, 1), (b, 1)))

    mesh = plsc.VectorSubcoreMesh(core_axis_name="c", subcore_axis_name="s")
    rows_per_worker = b // 32
    fa_bits = pl.kernel(
        functools.partial(_topk_sc_body, c=c, rows=rows_per_worker),
        mesh=mesh,
        out_type=jax.ShapeDtypeStruct((b, c), jnp.int32),
        scratch_types=[
            pltpu.VMEM((1, c), jnp.int32),
            pltpu.VMEM((1, c), jnp.int32),
            pltpu.VMEM((b,), jnp.int32),
        ],
        compiler_params=pltpu.CompilerParams(needs_layout_passes=False),
    )(fused_bits, k.reshape(b))
    fa = lax.bitcast_convert_type(fa_bits, jnp.float32)

    out1, out2 = pl.pallas_call(
        _apply_body,
        grid=(b,),
        in_specs=[
            pl.BlockSpec((1, hw, c), lambda i: (i, 0, 0)),
            pl.BlockSpec((1, 1, c), lambda i: (i, 0, 0)),
        ],
        out_specs=[
            pl.BlockSpec((1, hw, c), lambda i: (i, 0, 0)),
            pl.BlockSpec((1, hw, c), lambda i: (i, 0, 0)),
        ],
        out_shape=(
            jax.ShapeDtypeStruct((b, hw, c), jnp.float32),
            jax.ShapeDtypeStruct((b, hw, c), jnp.float32),
        ),
    )(xcl, fa.reshape(b, 1, c))

    def back(o):
        return o.reshape(b, h, w, c).transpose(0, 3, 1, 2)

    return back(out1), back(out2), pw.reshape(b, 3, 1, 1)
